# baseline (speedup 1.0000x reference)
import functools

import jax
import jax.numpy as jnp
from jax import lax
from jax.experimental import pallas as pl
from jax.experimental.pallas import tpu as pltpu
from jax.experimental.pallas import tpu_sc as plsc

NUM_P = 8
P_DROP = 0.1
NUM_GRAPHS = 128

_N = 10000
_E = 160000
_NT = NUM_P * _N
_PE = NUM_P * _E
_NSUB = 16
_NCORE = 2

_SC_MESH = plsc.VectorSubcoreMesh(core_axis_name="c", subcore_axis_name="s")


_CNT_ROWS = 10016
_IDG_PAD = 1280 * 128 - _E
_IDG_ROWS_PER_W = 1280 // (_NCORE * _NSUB)
_CNT_PER_TILE = _CNT_ROWS // _NSUB


def _indeg_body(di_hbm, ones_hbm, z_hbm, out_hbm, didx, ones_v, acc, ssem):
    c = lax.axis_index("c")
    s = lax.axis_index("s")
    w = c * _NSUB + s

    pltpu.sync_copy(ones_hbm, ones_v)
    pltpu.sync_copy(z_hbm.at[pl.ds(0, _CNT_PER_TILE)],
                    acc.at[pl.ds(s * _CNT_PER_TILE, _CNT_PER_TILE)])
    plsc.subcore_barrier()

    pltpu.sync_copy(di_hbm.at[pl.ds(w * _IDG_ROWS_PER_W, _IDG_ROWS_PER_W)],
                    didx)
    sds = [pltpu.async_copy(ones_v, acc.at[didx.at[j]], ssem, add=True)
           for j in range(_IDG_ROWS_PER_W)]
    for d in sds:
        d.wait()
    plsc.subcore_barrier()

    pltpu.sync_copy(acc.at[pl.ds(s * _CNT_PER_TILE, _CNT_PER_TILE)],
                    out_hbm.at[c].at[pl.ds(s * _CNT_PER_TILE, _CNT_PER_TILE)])


_indeg_call = pl.kernel(
    _indeg_body,
    out_type=jax.ShapeDtypeStruct((_NCORE, _CNT_ROWS, 16), jnp.float32),
    mesh=_SC_MESH,
    compiler_params=pltpu.CompilerParams(use_tc_tiling_on_sc=False),
    scratch_types=[
        pltpu.VMEM((_IDG_ROWS_PER_W, 128), jnp.int32),
        pltpu.VMEM((128, 16), jnp.float32),
        pltpu.VMEM_SHARED((_CNT_ROWS, 16), jnp.float32),
        pltpu.SemaphoreType.DMA,
    ],
)


_IDXROWS = _PE // 128
_ROWS_PER_TILE = _IDXROWS // _NSUB
_CHUNK_ROWS = 5
_NCHUNK = _ROWS_PER_TILE // _CHUNK_ROWS
_CHUNK_E = _CHUNK_ROWS * 128
_ACC_ROWS_PER_TILE = _NT // _NSUB


def _seg_body(hs_hbm, si_hbm, di_hbm, z_hbm, out_hbm,
              sidx0, didx0, sidx1, didx1, sidx2, didx2,
              rows0, rows1, rows2, acc, gsem, ssem):
    c = lax.axis_index("c")
    s = lax.axis_index("s")
    tr0 = s * _ROWS_PER_TILE
    CH = _CHUNK_ROWS
    NW = _NCHUNK
    SB = (sidx0, sidx1, sidx2)
    DB = (didx0, didx1, didx2)
    RB = (rows0, rows1, rows2)

    for fp in range(2):
        slab = c * 2 + fp
        hs_slab = hs_hbm.at[slab]

        def idx_load(w, r):
            rb = tr0 + w * CH
            pltpu.sync_copy(si_hbm.at[pl.ds(rb, CH)], SB[r])
            pltpu.sync_copy(di_hbm.at[pl.ds(rb, CH)], DB[r])

        def fire_g(r):
            for j in range(CH):
                pltpu.async_copy(hs_slab.at[SB[r].at[j]],
                                 RB[r].at[pl.ds(j * 128, 128)], gsem)

        def wait_g(r):
            for j in range(CH):
                pltpu.make_async_copy(hs_slab.at[SB[r].at[j]],
                                      RB[r].at[pl.ds(j * 128, 128)],
                                      gsem).wait()

        def fire_s(r):
            for j in range(CH):
                pltpu.async_copy(RB[r].at[pl.ds(j * 128, 128)],
                                 acc.at[DB[r].at[j]], ssem, add=True)

        def wait_s(r):
            for j in range(CH):
                pltpu.make_async_copy(RB[r].at[pl.ds(j * 128, 128)],
                                      acc.at[DB[r].at[j]], ssem).wait()

        pltpu.sync_copy(z_hbm, acc.at[pl.ds(s * _ACC_ROWS_PER_TILE,
                                            _ACC_ROWS_PER_TILE)])
        plsc.subcore_barrier()

        def body(w, r, first, last):
            if not first:
                wait_s(r)
            idx_load(w, r)
            fire_g(r)
            if not last:
                pr = (r - 1) % 3
                wait_g(pr)
                fire_s(pr)

        body(0, 0, True, True)
        body(1, 1, True, False)
        body(2, 2, True, False)

        def step(i, carry):
            w0 = 3 * i
            for r in range(3):
                body(w0 + r, r, False, False)
            return carry

        lax.fori_loop(1, (NW - 2) // 3, step, 0)

        body(NW - 2, 0, False, False)
        body(NW - 1, 1, False, False)
        wait_g(1)
        fire_s(1)
        wait_s(2)
        wait_s(0)
        wait_s(1)

        plsc.subcore_barrier()
        pltpu.sync_copy(acc.at[pl.ds(s * _ACC_ROWS_PER_TILE,
                                     _ACC_ROWS_PER_TILE)],
                        out_hbm.at[slab].at[pl.ds(s * _ACC_ROWS_PER_TILE,
                                                  _ACC_ROWS_PER_TILE)])
        plsc.subcore_barrier()


_seg_call = pl.kernel(
    _seg_body,
    out_type=jax.ShapeDtypeStruct((4, _NT, 16), jnp.float32),
    mesh=_SC_MESH,
    compiler_params=pltpu.CompilerParams(use_tc_tiling_on_sc=False),
    scratch_types=[
        pltpu.VMEM((_CHUNK_ROWS, 128), jnp.int32),
        pltpu.VMEM((_CHUNK_ROWS, 128), jnp.int32),
        pltpu.VMEM((_CHUNK_ROWS, 128), jnp.int32),
        pltpu.VMEM((_CHUNK_ROWS, 128), jnp.int32),
        pltpu.VMEM((_CHUNK_ROWS, 128), jnp.int32),
        pltpu.VMEM((_CHUNK_ROWS, 128), jnp.int32),
        pltpu.VMEM((_CHUNK_E, 16), jnp.float32),
        pltpu.VMEM((_CHUNK_E, 16), jnp.float32),
        pltpu.VMEM((_CHUNK_E, 16), jnp.float32),
        pltpu.VMEM_SHARED((_NT, 16), jnp.float32),
        pltpu.SemaphoreType.DMA,
        pltpu.SemaphoreType.DMA,
    ],
)


def _final_kernel(z_ref, oneh_ref, fcb_ref, out_ref):
    out_ref[...] = jnp.dot(oneh_ref[...].T, z_ref[...],
                           preferred_element_type=jnp.float32) + fcb_ref[...]


def kernel(x, edge_index, batch, convW0, convb0, bnG0, bnB0, convW1, convb1,
           bnG1, bnB1, convW2, convb2, bnG2, bnB2, convW3, convb3, bnG3, bnB3,
           fcW0, fcb0, fcW1, fcb1, fcW2, fcb2, fcW3, fcb3, fcW4, fcb4):
    N, D = x.shape
    P = NUM_P
    n_tot = P * N

    convs = [(convW0, convb0), (convW1, convb1), (convW2, convb2), (convW3, convb3)]
    bns = [(bnG0, bnB0), (bnG1, bnB1), (bnG2, bnB2), (bnG3, bnB3)]
    fcs = [(fcW0, fcb0), (fcW1, fcb1), (fcW2, fcb2), (fcW3, fcb3), (fcW4, fcb4)]

    drop = jax.random.bernoulli(jax.random.key(42), P_DROP, (P, N))
    keep = jnp.where(drop, 0.0, 1.0).astype(x.dtype)

    src = edge_index[0]
    dst = edge_index[1]
    off = jnp.max(edge_index) + 1

    shift = (jnp.arange(P, dtype=src.dtype) * off)[:, None]
    si2d = (src[None, :] + shift).reshape(_IDXROWS, 128)
    di2d = (dst[None, :] + shift).reshape(_IDXROWS, 128)

    zeros_slab = jnp.zeros((_ACC_ROWS_PER_TILE, 16), jnp.float32)

    dstp = jnp.concatenate(
        [dst, jnp.full((_IDG_PAD,), _N, jnp.int32)]).reshape(1280, 128)
    cnt = _indeg_call(dstp, jnp.ones((128, 16), jnp.float32), zeros_slab)
    indeg = (cnt[0] + cnt[1])[:_N, 0]

    big = jnp.concatenate([jnp.zeros((n_tot - N,), jnp.float32), indeg,
                           jnp.zeros((n_tot - N,), jnp.float32)])
    deg = jnp.ones((n_tot,), jnp.float32)
    for p in range(P):
        deg = deg + lax.dynamic_slice(big, ((n_tot - N) - p * off,), (n_tot,))
    dinv = deg ** -0.5

    keep_mean = keep.mean(axis=0)
    z = (keep_mean[:, None] * x) @ fcs[0][0]

    h0 = x @ convs[0][0]
    hs = (keep[:, :, None] * h0[None]).reshape(n_tot, -1) * dinv[:, None]

    for i in range(4):
        if i > 0:
            hs = (xf @ convs[i][0]) * dinv[:, None]
        hs_slabs = hs.reshape(n_tot, 4, 16).transpose(1, 0, 2)
        seg = _seg_call(hs_slabs, si2d, di2d, zeros_slab
                        ).transpose(1, 0, 2).reshape(n_tot, 64)
        u = dinv[:, None] * (seg + hs) + convs[i][1]
        mu = u.mean(axis=0)
        var = ((u - mu) ** 2).mean(axis=0)
        g, bt = bns[i]
        xf = jax.nn.relu((u - mu) * lax.rsqrt(var + 1e-5) * g + bt)
        m = xf.reshape(P, N, -1).mean(axis=0)
        z = z + m @ fcs[i + 1][0]

    oneh = (batch[:, None] == jnp.arange(NUM_GRAPHS, dtype=batch.dtype)[None, :]
            ).astype(jnp.float32)
    fcb_sum = fcb0 + fcb1 + fcb2 + fcb3 + fcb4

    out = pl.pallas_call(
        _final_kernel,
        out_shape=jax.ShapeDtypeStruct((NUM_GRAPHS, 2), jnp.float32),
    )(z, oneh, jnp.broadcast_to(fcb_sum[None, :], (NUM_GRAPHS, 2)))
    return out


# speedup vs baseline: 1.0145x; 1.0145x over previous
import functools

import jax
import jax.numpy as jnp
from jax import lax
from jax.experimental import pallas as pl
from jax.experimental.pallas import tpu as pltpu
from jax.experimental.pallas import tpu_sc as plsc

NUM_P = 8
P_DROP = 0.1
NUM_GRAPHS = 128

_N = 10000
_E = 160000
_NT = NUM_P * _N
_PE = NUM_P * _E
_NSUB = 16
_NCORE = 2

_SC_MESH = plsc.VectorSubcoreMesh(core_axis_name="c", subcore_axis_name="s")


_CNT_ROWS = 10016
_IDG_PAD = 1280 * 128 - _E
_IDG_ROWS_PER_W = 1280 // (_NCORE * _NSUB)
_CNT_PER_TILE = _CNT_ROWS // _NSUB


def _indeg_body(di_hbm, ones_hbm, z_hbm, out_hbm, didx, ones_v, acc, ssem):
    c = lax.axis_index("c")
    s = lax.axis_index("s")
    w = c * _NSUB + s

    pltpu.sync_copy(ones_hbm, ones_v)
    pltpu.sync_copy(z_hbm.at[pl.ds(0, _CNT_PER_TILE)],
                    acc.at[pl.ds(s * _CNT_PER_TILE, _CNT_PER_TILE)])
    plsc.subcore_barrier()

    pltpu.sync_copy(di_hbm.at[pl.ds(w * _IDG_ROWS_PER_W, _IDG_ROWS_PER_W)],
                    didx)
    sds = [pltpu.async_copy(ones_v, acc.at[didx.at[j]], ssem, add=True)
           for j in range(_IDG_ROWS_PER_W)]
    for d in sds:
        d.wait()
    plsc.subcore_barrier()

    pltpu.sync_copy(acc.at[pl.ds(s * _CNT_PER_TILE, _CNT_PER_TILE)],
                    out_hbm.at[c].at[pl.ds(s * _CNT_PER_TILE, _CNT_PER_TILE)])


_indeg_call = pl.kernel(
    _indeg_body,
    out_type=jax.ShapeDtypeStruct((_NCORE, _CNT_ROWS, 16), jnp.float32),
    mesh=_SC_MESH,
    compiler_params=pltpu.CompilerParams(use_tc_tiling_on_sc=False),
    scratch_types=[
        pltpu.VMEM((_IDG_ROWS_PER_W, 128), jnp.int32),
        pltpu.VMEM((128, 16), jnp.float32),
        pltpu.VMEM_SHARED((_CNT_ROWS, 16), jnp.float32),
        pltpu.SemaphoreType.DMA,
    ],
)


_IDXROWS = _PE // 128
_ROWS_PER_TILE = _IDXROWS // _NSUB
_CHUNK_ROWS = 5
_NCHUNK = _ROWS_PER_TILE // _CHUNK_ROWS
_CHUNK_E = _CHUNK_ROWS * 128
_ACC_ROWS_PER_TILE = _NT // _NSUB


def _seg_body(hs_hbm, si_hbm, di_hbm, z_hbm, out_hbm,
              sidx0, didx0, sidx1, didx1, rows0, rows1, acc, gsem, ssem):
    c = lax.axis_index("c")
    s = lax.axis_index("s")
    tr0 = s * _ROWS_PER_TILE
    CH = _CHUNK_ROWS
    NW = _NCHUNK

    for fp in range(2):
        slab = c * 2 + fp
        hs_slab = hs_hbm.at[slab]

        def idx_load(w, sb, db):
            rb = tr0 + w * CH
            pltpu.sync_copy(si_hbm.at[pl.ds(rb, CH)], sb)
            pltpu.sync_copy(di_hbm.at[pl.ds(rb, CH)], db)

        def fire_g(sb, rbuf):
            return [pltpu.async_copy(hs_slab.at[sb.at[j]],
                                     rbuf.at[pl.ds(j * 128, 128)], gsem)
                    for j in range(CH)]

        def wait_g(sb, rbuf):
            for j in range(CH):
                pltpu.make_async_copy(hs_slab.at[sb.at[j]],
                                      rbuf.at[pl.ds(j * 128, 128)],
                                      gsem).wait()

        def fire_s(db, rbuf):
            return [pltpu.async_copy(rbuf.at[pl.ds(j * 128, 128)],
                                     acc.at[db.at[j]], ssem, add=True)
                    for j in range(CH)]

        def wait_s(db, rbuf):
            for j in range(CH):
                pltpu.make_async_copy(rbuf.at[pl.ds(j * 128, 128)],
                                      acc.at[db.at[j]], ssem).wait()

        pltpu.sync_copy(z_hbm, acc.at[pl.ds(s * _ACC_ROWS_PER_TILE,
                                            _ACC_ROWS_PER_TILE)])
        plsc.subcore_barrier()

        idx_load(0, sidx0, didx0)
        fire_g(sidx0, rows0)
        idx_load(1, sidx1, didx1)
        wait_g(sidx0, rows0)
        s0 = fire_s(didx0, rows0)
        fire_g(sidx1, rows1)
        for d in s0:
            d.wait()
        idx_load(2, sidx0, didx0)
        fire_g(sidx0, rows0)
        wait_g(sidx1, rows1)
        fire_s(didx1, rows1)

        def step(i, carry):
            A = 2 * i
            wait_s(didx1, rows1)
            idx_load(A + 1, sidx1, didx1)
            fire_g(sidx1, rows1)
            wait_g(sidx0, rows0)
            sA = fire_s(didx0, rows0)
            for d in sA:
                d.wait()
            idx_load(A + 2, sidx0, didx0)
            fire_g(sidx0, rows0)
            wait_g(sidx1, rows1)
            fire_s(didx1, rows1)
            return carry

        lax.fori_loop(1, (NW - 1) // 2, step, 0)

        wait_s(didx1, rows1)
        wait_g(sidx0, rows0)
        sl = fire_s(didx0, rows0)
        for d in sl:
            d.wait()

        plsc.subcore_barrier()
        pltpu.sync_copy(acc.at[pl.ds(s * _ACC_ROWS_PER_TILE,
                                     _ACC_ROWS_PER_TILE)],
                        out_hbm.at[slab].at[pl.ds(s * _ACC_ROWS_PER_TILE,
                                                  _ACC_ROWS_PER_TILE)])
        plsc.subcore_barrier()


_seg_call = pl.kernel(
    _seg_body,
    out_type=jax.ShapeDtypeStruct((4, _NT, 16), jnp.float32),
    mesh=_SC_MESH,
    compiler_params=pltpu.CompilerParams(use_tc_tiling_on_sc=False),
    scratch_types=[
        pltpu.VMEM((_CHUNK_ROWS, 128), jnp.int32),
        pltpu.VMEM((_CHUNK_ROWS, 128), jnp.int32),
        pltpu.VMEM((_CHUNK_ROWS, 128), jnp.int32),
        pltpu.VMEM((_CHUNK_ROWS, 128), jnp.int32),
        pltpu.VMEM((_CHUNK_E, 16), jnp.float32),
        pltpu.VMEM((_CHUNK_E, 16), jnp.float32),
        pltpu.VMEM_SHARED((_NT, 16), jnp.float32),
        pltpu.SemaphoreType.DMA,
        pltpu.SemaphoreType.DMA,
    ],
)


def _final_kernel(z_ref, oneh_ref, fcb_ref, out_ref):
    out_ref[...] = jnp.dot(oneh_ref[...].T, z_ref[...],
                           preferred_element_type=jnp.float32) + fcb_ref[...]


def kernel(x, edge_index, batch, convW0, convb0, bnG0, bnB0, convW1, convb1,
           bnG1, bnB1, convW2, convb2, bnG2, bnB2, convW3, convb3, bnG3, bnB3,
           fcW0, fcb0, fcW1, fcb1, fcW2, fcb2, fcW3, fcb3, fcW4, fcb4):
    N, D = x.shape
    P = NUM_P
    n_tot = P * N

    convs = [(convW0, convb0), (convW1, convb1), (convW2, convb2), (convW3, convb3)]
    bns = [(bnG0, bnB0), (bnG1, bnB1), (bnG2, bnB2), (bnG3, bnB3)]
    fcs = [(fcW0, fcb0), (fcW1, fcb1), (fcW2, fcb2), (fcW3, fcb3), (fcW4, fcb4)]

    drop = jax.random.bernoulli(jax.random.key(42), P_DROP, (P, N))
    keep = jnp.where(drop, 0.0, 1.0).astype(x.dtype)

    src = edge_index[0]
    dst = edge_index[1]
    off = jnp.max(edge_index) + 1

    shift = (jnp.arange(P, dtype=src.dtype) * off)[:, None]
    si2d = (src[None, :] + shift).reshape(_IDXROWS, 128)
    di2d = (dst[None, :] + shift).reshape(_IDXROWS, 128)

    zeros_slab = jnp.zeros((_ACC_ROWS_PER_TILE, 16), jnp.float32)

    dstp = jnp.concatenate(
        [dst, jnp.full((_IDG_PAD,), _N, jnp.int32)]).reshape(1280, 128)
    cnt = _indeg_call(dstp, jnp.ones((128, 16), jnp.float32), zeros_slab)
    indeg = (cnt[0] + cnt[1])[:_N, 0]

    big = jnp.concatenate([jnp.zeros((n_tot - N,), jnp.float32), indeg,
                           jnp.zeros((n_tot - N,), jnp.float32)])
    deg = jnp.ones((n_tot,), jnp.float32)
    for p in range(P):
        deg = deg + lax.dynamic_slice(big, ((n_tot - N) - p * off,), (n_tot,))
    dinv = deg ** -0.5

    keep_mean = keep.mean(axis=0)
    z = (keep_mean[:, None] * x) @ fcs[0][0]

    h0 = x @ convs[0][0]
    hs = (keep[:, :, None] * h0[None]).reshape(n_tot, -1) * dinv[:, None]

    for i in range(4):
        if i > 0:
            hs = (xf @ convs[i][0]) * dinv[:, None]
        hs_slabs = hs.reshape(n_tot, 4, 16).transpose(1, 0, 2)
        seg = _seg_call(hs_slabs, si2d, di2d, zeros_slab
                        ).transpose(1, 0, 2).reshape(n_tot, 64)
        u = dinv[:, None] * (seg + hs) + convs[i][1]
        mu = u.mean(axis=0)
        var = ((u - mu) ** 2).mean(axis=0)
        g, bt = bns[i]
        xf = jax.nn.relu((u - mu) * lax.rsqrt(var + 1e-5) * g + bt)
        m = xf.reshape(P, N, -1).mean(axis=0)
        z = z + m @ fcs[i + 1][0]

    oneh = (batch[:, None] == jnp.arange(NUM_GRAPHS, dtype=batch.dtype)[None, :]
            ).astype(jnp.float32)
    fcb_sum = fcb0 + fcb1 + fcb2 + fcb3 + fcb4

    out = pl.pallas_call(
        _final_kernel,
        out_shape=jax.ShapeDtypeStruct((NUM_GRAPHS, 2), jnp.float32),
    )(z, oneh, jnp.broadcast_to(fcb_sum[None, :], (NUM_GRAPHS, 2)))
    return out


# speedup vs baseline: 1.0154x; 1.0009x over previous
import functools

import jax
import jax.numpy as jnp
from jax import lax
from jax.experimental import pallas as pl
from jax.experimental.pallas import tpu as pltpu
from jax.experimental.pallas import tpu_sc as plsc

NUM_P = 8
P_DROP = 0.1
NUM_GRAPHS = 128

_N = 10000
_E = 160000
_NT = NUM_P * _N
_PE = NUM_P * _E
_NSUB = 16
_NCORE = 2

_SC_MESH = plsc.VectorSubcoreMesh(core_axis_name="c", subcore_axis_name="s")


_CNT_ROWS = 10016
_IDG_PAD = 1280 * 128 - _E
_IDG_ROWS_PER_W = 1280 // (_NCORE * _NSUB)
_CNT_PER_TILE = _CNT_ROWS // _NSUB


def _indeg_body(di_hbm, ones_hbm, z_hbm, out_hbm, didx, ones_v, acc, ssem):
    c = lax.axis_index("c")
    s = lax.axis_index("s")
    w = c * _NSUB + s

    pltpu.sync_copy(ones_hbm, ones_v)
    pltpu.sync_copy(z_hbm.at[pl.ds(0, _CNT_PER_TILE)],
                    acc.at[pl.ds(s * _CNT_PER_TILE, _CNT_PER_TILE)])
    plsc.subcore_barrier()

    pltpu.sync_copy(di_hbm.at[pl.ds(w * _IDG_ROWS_PER_W, _IDG_ROWS_PER_W)],
                    didx)
    sds = [pltpu.async_copy(ones_v, acc.at[didx.at[j]], ssem, add=True)
           for j in range(_IDG_ROWS_PER_W)]
    for d in sds:
        d.wait()
    plsc.subcore_barrier()

    pltpu.sync_copy(acc.at[pl.ds(s * _CNT_PER_TILE, _CNT_PER_TILE)],
                    out_hbm.at[c].at[pl.ds(s * _CNT_PER_TILE, _CNT_PER_TILE)])


_indeg_call = pl.kernel(
    _indeg_body,
    out_type=jax.ShapeDtypeStruct((_NCORE, _CNT_ROWS, 16), jnp.float32),
    mesh=_SC_MESH,
    compiler_params=pltpu.CompilerParams(use_tc_tiling_on_sc=False),
    scratch_types=[
        pltpu.VMEM((_IDG_ROWS_PER_W, 128), jnp.int32),
        pltpu.VMEM((128, 16), jnp.float32),
        pltpu.VMEM_SHARED((_CNT_ROWS, 16), jnp.float32),
        pltpu.SemaphoreType.DMA,
    ],
)


_IDXROWS = _PE // 128
_ROWS_PER_TILE = _IDXROWS // _NSUB
_CHUNK_ROWS = 5
_NCHUNK = _ROWS_PER_TILE // _CHUNK_ROWS
_CHUNK_E = _CHUNK_ROWS * 128
_ACC_ROWS_PER_TILE = _NT // _NSUB


def _seg_body(hs_hbm, si_hbm, di_hbm, z_hbm, out_hbm,
              sidx0, didx0, sidx1, didx1, rows0, rows1, acc, gsem, ssem):
    c = lax.axis_index("c")
    s = lax.axis_index("s")
    tr0 = s * _ROWS_PER_TILE
    CH = _CHUNK_ROWS
    NW = _NCHUNK

    for fp in range(2):
        slab = c * 2 + fp
        hs_slab = hs_hbm.at[slab]

        def idx_load(w, sb, db):
            rb = tr0 + w * CH
            pltpu.sync_copy(si_hbm.at[pl.ds(rb, CH)], sb)
            pltpu.sync_copy(di_hbm.at[pl.ds(rb, CH)], db)

        def fire_g(sb, rbuf):
            return [pltpu.async_copy(hs_slab.at[sb.at[j]],
                                     rbuf.at[pl.ds(j * 128, 128)], gsem)
                    for j in range(CH)]

        def wait_g(sb, rbuf):
            for j in range(CH):
                pltpu.make_async_copy(hs_slab.at[sb.at[j]],
                                      rbuf.at[pl.ds(j * 128, 128)],
                                      gsem).wait()

        def fire_s(db, rbuf):
            return [pltpu.async_copy(rbuf.at[pl.ds(j * 128, 128)],
                                     acc.at[db.at[j]], ssem, add=True)
                    for j in range(CH)]

        def wait_s(db, rbuf):
            for j in range(CH):
                pltpu.make_async_copy(rbuf.at[pl.ds(j * 128, 128)],
                                      acc.at[db.at[j]], ssem).wait()

        pltpu.sync_copy(z_hbm, acc.at[pl.ds(s * _ACC_ROWS_PER_TILE,
                                            _ACC_ROWS_PER_TILE)])
        plsc.subcore_barrier()

        idx_load(0, sidx0, didx0)
        fire_g(sidx0, rows0)
        idx_load(1, sidx1, didx1)
        wait_g(sidx0, rows0)
        s0 = fire_s(didx0, rows0)
        fire_g(sidx1, rows1)
        for d in s0:
            d.wait()
        idx_load(2, sidx0, didx0)
        fire_g(sidx0, rows0)
        wait_g(sidx1, rows1)
        fire_s(didx1, rows1)

        def step(i, carry):
            A = 2 * i
            wait_s(didx1, rows1)
            idx_load(A + 1, sidx1, didx1)
            fire_g(sidx1, rows1)
            wait_g(sidx0, rows0)
            sA = fire_s(didx0, rows0)
            for d in sA:
                d.wait()
            idx_load(A + 2, sidx0, didx0)
            fire_g(sidx0, rows0)
            wait_g(sidx1, rows1)
            fire_s(didx1, rows1)
            return carry

        lax.fori_loop(1, (NW - 1) // 2, step, 0)

        wait_s(didx1, rows1)
        wait_g(sidx0, rows0)
        sl = fire_s(didx0, rows0)
        for d in sl:
            d.wait()

        plsc.subcore_barrier()
        pltpu.sync_copy(acc.at[pl.ds(s * _ACC_ROWS_PER_TILE,
                                     _ACC_ROWS_PER_TILE)],
                        out_hbm.at[slab].at[pl.ds(s * _ACC_ROWS_PER_TILE,
                                                  _ACC_ROWS_PER_TILE)])
        plsc.subcore_barrier()


_seg_call = pl.kernel(
    _seg_body,
    out_type=jax.ShapeDtypeStruct((4, _NT, 16), jnp.float32),
    mesh=_SC_MESH,
    compiler_params=pltpu.CompilerParams(use_tc_tiling_on_sc=False),
    scratch_types=[
        pltpu.VMEM((_CHUNK_ROWS, 128), jnp.int32),
        pltpu.VMEM((_CHUNK_ROWS, 128), jnp.int32),
        pltpu.VMEM((_CHUNK_ROWS, 128), jnp.int32),
        pltpu.VMEM((_CHUNK_ROWS, 128), jnp.int32),
        pltpu.VMEM((_CHUNK_E, 16), jnp.float32),
        pltpu.VMEM((_CHUNK_E, 16), jnp.float32),
        pltpu.VMEM_SHARED((_NT, 16), jnp.float32),
        pltpu.SemaphoreType.DMA,
        pltpu.SemaphoreType.DMA,
    ],
)


def _final_kernel(z_ref, oneh_ref, fcb_ref, out_ref):
    out_ref[...] = jnp.dot(oneh_ref[...].T, z_ref[...],
                           preferred_element_type=jnp.float32) + fcb_ref[...]


def kernel(x, edge_index, batch, convW0, convb0, bnG0, bnB0, convW1, convb1,
           bnG1, bnB1, convW2, convb2, bnG2, bnB2, convW3, convb3, bnG3, bnB3,
           fcW0, fcb0, fcW1, fcb1, fcW2, fcb2, fcW3, fcb3, fcW4, fcb4):
    N, D = x.shape
    P = NUM_P
    n_tot = P * N

    convs = [(convW0, convb0), (convW1, convb1), (convW2, convb2), (convW3, convb3)]
    bns = [(bnG0, bnB0), (bnG1, bnB1), (bnG2, bnB2), (bnG3, bnB3)]
    fcs = [(fcW0, fcb0), (fcW1, fcb1), (fcW2, fcb2), (fcW3, fcb3), (fcW4, fcb4)]

    drop = jax.random.bernoulli(jax.random.key(42), P_DROP, (P, N))
    keep = jnp.where(drop, 0.0, 1.0).astype(x.dtype)

    src = edge_index[0]
    dst = edge_index[1]
    off = jnp.max(edge_index) + 1

    shift = (jnp.arange(P, dtype=src.dtype) * off)[:, None]
    si2d = (src[None, :] + shift).reshape(_IDXROWS, 128)
    di2d = (dst[None, :] + shift).reshape(_IDXROWS, 128)

    zeros_slab = jnp.zeros((_ACC_ROWS_PER_TILE, 16), jnp.float32)

    dstp = jnp.concatenate(
        [dst, jnp.full((_IDG_PAD,), _N, jnp.int32)]).reshape(1280, 128)
    cnt = _indeg_call(dstp, jnp.ones((128, 16), jnp.float32), zeros_slab)
    indeg = (cnt[0] + cnt[1])[:_N, 0]

    big = jnp.concatenate([jnp.zeros((n_tot - N,), jnp.float32), indeg,
                           jnp.zeros((n_tot - N,), jnp.float32)])
    deg = jnp.ones((n_tot,), jnp.float32)
    for p in range(P):
        deg = deg + lax.dynamic_slice(big, ((n_tot - N) - p * off,), (n_tot,))
    dinv = deg ** -0.5

    keep_mean = keep.mean(axis=0)
    z = (keep_mean[:, None] * x) @ fcs[0][0]

    h0 = x @ convs[0][0]
    hs = (keep[:, :, None] * h0[None]).reshape(n_tot, -1) * dinv[:, None]

    for i in range(4):
        if i > 0:
            hs = (xf @ convs[i][0]) * dinv[:, None]
        hs_slabs = hs.reshape(n_tot, 4, 16).transpose(1, 0, 2)
        seg = _seg_call(hs_slabs, si2d, di2d, zeros_slab
                        ).transpose(1, 0, 2).reshape(n_tot, 64)
        u = dinv[:, None] * (seg + hs) + convs[i][1]
        mu = u.mean(axis=0)
        var = ((u - mu) ** 2).mean(axis=0)
        g, bt = bns[i]
        xf = jax.nn.relu((u - mu) / jnp.sqrt(var + 1e-5) * g + bt)
        m = xf.reshape(P, N, -1).mean(axis=0)
        z = z + m @ fcs[i + 1][0]

    oneh = (batch[:, None] == jnp.arange(NUM_GRAPHS, dtype=batch.dtype)[None, :]
            ).astype(jnp.float32)
    fcb_sum = fcb0 + fcb1 + fcb2 + fcb3 + fcb4

    out = pl.pallas_call(
        _final_kernel,
        out_shape=jax.ShapeDtypeStruct((NUM_GRAPHS, 2), jnp.float32),
    )(z, oneh, jnp.broadcast_to(fcb_sum[None, :], (NUM_GRAPHS, 2)))
    return out
